# baseline (speedup 1.0000x reference)
import functools

import jax
import jax.numpy as jnp
from jax import lax
from jax.experimental import pallas as pl
from jax.experimental.pallas import tpu as pltpu
from jax.experimental.pallas import tpu_sc as plsc

NN = 10000
ACC_ROWS = 10112
EP = 327680
LANES = 128
NTILES = 16
G = 64
NC = 10
RB = 2000
EPS_BN = 1e-5
F32 = jnp.float32


IDXC = 4
BSTEPS = 2 * IDXC


def _sc_edge_loop(h_ref, eb, row0, idx_a, idx_b, rows, acc, gsems, ssems,
                  isems, nrows):
    nbody = nrows // BSTEPS

    def drain_rows(sem, b):
        pltpu.make_async_copy(h_ref.at[pl.ds(0, LANES)], rows[b], sem).wait()

    def drain_idx(sem, ix):
        pltpu.make_async_copy(eb.at[pl.ds(0, IDXC)], ix, sem).wait()

    def body(base, first, last):
        if first:
            pltpu.sync_copy(eb.at[pl.ds(base, IDXC)], idx_a)
            pltpu.async_copy(h_ref.at[idx_a.at[0, 0]], rows[0], gsems[0])
        for j in range(BSTEPS):
            b = j % 2
            nb = 1 - b
            if not (first and j == 0):
                drain_rows(ssems[nb], nb)
            if j == 0:
                pltpu.async_copy(eb.at[pl.ds(base + IDXC, IDXC)], idx_b,
                                 isems[1])
            if j == IDXC and not last:
                pltpu.async_copy(eb.at[pl.ds(base + BSTEPS, IDXC)], idx_a,
                                 isems[0])
            if j + 1 < BSTEPS:
                nxt = j + 1
                if nxt == IDXC:
                    drain_idx(isems[1], idx_b)
                ix = idx_a if nxt < IDXC else idx_b
                pltpu.async_copy(h_ref.at[ix.at[nxt % IDXC, 0]], rows[nb],
                                 gsems[nb])
            elif not last:
                drain_idx(isems[0], idx_a)
                pltpu.async_copy(h_ref.at[idx_a.at[0, 0]], rows[nb],
                                 gsems[nb])
            drain_rows(gsems[b], b)
            ix = idx_a if j < IDXC else idx_b
            pltpu.async_copy(rows[b], acc.at[ix.at[j % IDXC, 1]], ssems[b],
                             add=True)
        if last:
            drain_rows(ssems[1], 1)

    body(row0, True, False)

    def mid(k2, carry):
        body(row0 + k2 * BSTEPS, False, False)
        return carry

    lax.fori_loop(1, nbody - 1, mid, 0)
    body(row0 + (nbody - 1) * BSTEPS, False, True)


def _sc_zero_acc(zeros_hbm, acc, s):
    zc = ACC_ROWS // NTILES
    pltpu.sync_copy(zeros_hbm.at[pl.ds(s * zc, zc)], acc.at[pl.ds(s * zc, zc)])
    plsc.subcore_barrier()


def _sc_copy_out(acc, agg0, agg1, c, s):
    plsc.subcore_barrier()
    oc = ACC_ROWS // NTILES

    @pl.when(c == 0)
    def _():
        pltpu.sync_copy(acc.at[pl.ds(s * oc, oc)], agg0.at[pl.ds(s * oc, oc)])

    @pl.when(c == 1)
    def _():
        pltpu.sync_copy(acc.at[pl.ds(s * oc, oc)], agg1.at[pl.ds(s * oc, oc)])


def _make_sc_l1():
    nrows = EP // 128 // (2 * NTILES)
    mesh = plsc.VectorSubcoreMesh(core_axis_name="c", subcore_axis_name="s")

    @functools.partial(
        pl.kernel,
        out_type=[jax.ShapeDtypeStruct((ACC_ROWS, 128), F32),
                  jax.ShapeDtypeStruct((ACC_ROWS, 128), F32)],
        mesh=mesh,
        scratch_types=[
            pltpu.VMEM((IDXC, 2, LANES), jnp.int32),
            pltpu.VMEM((IDXC, 2, LANES), jnp.int32),
            pltpu.VMEM((LANES, 128), F32),
            pltpu.VMEM((LANES, 128), F32),
            pltpu.VMEM_SHARED((ACC_ROWS, 128), F32),
            pltpu.SemaphoreType.DMA,
            pltpu.SemaphoreType.DMA,
            pltpu.SemaphoreType.DMA,
            pltpu.SemaphoreType.DMA,
            pltpu.SemaphoreType.DMA,
            pltpu.SemaphoreType.DMA,
        ],
    )
    def sc_l1(h_hbm, eb, zeros_hbm, agg0, agg1,
              idx_a, idx_b, rows_a, rows_b, acc,
              gs0, gs1, ss0, ss1, is0, is1):
        c = lax.axis_index("c")
        s = lax.axis_index("s")
        _sc_zero_acc(zeros_hbm, acc, s)
        row0 = (c * NTILES + s) * nrows
        _sc_edge_loop(h_hbm, eb, row0, idx_a, idx_b, (rows_a, rows_b), acc,
                      (gs0, gs1), (ss0, ss1), (is0, is1), nrows)
        _sc_copy_out(acc, agg0, agg1, c, s)

    return sc_l1


def _make_sc_l23():
    nrows = EP // 128 // NTILES
    mesh = plsc.VectorSubcoreMesh(core_axis_name="c", subcore_axis_name="s")

    @functools.partial(
        pl.kernel,
        out_type=[jax.ShapeDtypeStruct((ACC_ROWS, 128), F32),
                  jax.ShapeDtypeStruct((ACC_ROWS, 128), F32)],
        mesh=mesh,
        scratch_types=[
            pltpu.VMEM((IDXC, 2, LANES), jnp.int32),
            pltpu.VMEM((IDXC, 2, LANES), jnp.int32),
            pltpu.VMEM((LANES, 128), F32),
            pltpu.VMEM((LANES, 128), F32),
            pltpu.VMEM_SHARED((ACC_ROWS, 128), F32),
            pltpu.SemaphoreType.DMA,
            pltpu.SemaphoreType.DMA,
            pltpu.SemaphoreType.DMA,
            pltpu.SemaphoreType.DMA,
            pltpu.SemaphoreType.DMA,
            pltpu.SemaphoreType.DMA,
        ],
    )
    def sc_l23(h0_hbm, h1_hbm, eb, zeros_hbm, agg0, agg1,
               idx_a, idx_b, rows_a, rows_b, acc,
               gs0, gs1, ss0, ss1, is0, is1):
        c = lax.axis_index("c")
        s = lax.axis_index("s")
        _sc_zero_acc(zeros_hbm, acc, s)
        row0 = s * nrows

        @pl.when(c == 0)
        def _():
            _sc_edge_loop(h0_hbm, eb, row0, idx_a, idx_b, (rows_a, rows_b),
                          acc, (gs0, gs1), (ss0, ss1), (is0, is1), nrows)

        @pl.when(c == 1)
        def _():
            _sc_edge_loop(h1_hbm, eb, row0, idx_a, idx_b, (rows_a, rows_b),
                          acc, (gs0, gs1), (ss0, ss1), (is0, is1), nrows)

        _sc_copy_out(acc, agg0, agg1, c, s)

    return sc_l23


def _accum_stats(i, y_blk, s_ref, q_ref):
    s_p = jnp.sum(y_blk, axis=0, keepdims=True)
    q_p = jnp.sum(y_blk * y_blk, axis=0, keepdims=True)

    @pl.when(i == 0)
    def _():
        s_ref[...] = s_p
        q_ref[...] = q_p

    @pl.when(i > 0)
    def _():
        s_ref[...] = s_ref[...] + s_p
        q_ref[...] = q_ref[...] + q_p


def _bn_coeffs(s_ref, q_ref, g_ref, be_ref):
    m = s_ref[...] / NN
    v = q_ref[...] / NN - m * m
    a = g_ref[...] * lax.rsqrt(v + EPS_BN)
    c = be_ref[...] - m * a
    return a, c


NP5 = NN // RB


def _c3_phase12(i, h0, h1, a0, a1, wa, wb, b1, w2, b2, g1, be1,
                yin1, y1, s1, q1, y2, s2, q2):
    @pl.when(i < NP5)
    def _():
        xa = h0[...] + a0[...]
        xb = h1[...] + a1[...]
        blk = (jnp.dot(xa, wa[...], preferred_element_type=F32)
               + jnp.dot(xb, wb[...], preferred_element_type=F32)
               + b1[...])
        y1[...] = blk
        _accum_stats(i, blk, s1, q1)

    @pl.when(jnp.logical_and(i >= NP5, i < 2 * NP5))
    def _():
        a, c = _bn_coeffs(s1, q1, g1, be1)
        yb = yin1[...]
        y1[...] = yb
        x = jnp.maximum(yb * a + c, 0.0)
        blk = jnp.dot(x, w2[...], preferred_element_type=F32) + b2[...]
        y2[...] = blk
        _accum_stats(i - NP5, blk, s2, q2)


def _c3_mid_body(h0, h1, a0, a1, wa, wb, b1, w2, b2, g1, be1, g2, be2,
                 yin1, yin2, y1, s1, q1, y2, s2, q2, hp0, hp1):
    i = pl.program_id(0)
    _c3_phase12(i, h0, h1, a0, a1, wa, wb, b1, w2, b2, g1, be1,
                yin1, y1, s1, q1, y2, s2, q2)

    @pl.when(i >= 2 * NP5)
    def _():
        a, c = _bn_coeffs(s2, q2, g2, be2)
        yb = yin2[...]
        y2[...] = yb
        h = jnp.maximum(yb * a + c, 0.0)
        hp0[...] = h[:, :128]
        hp1[...] = h[:, 128:]


def _c3_last_body(h0, h1, a0, a1, wa, wb, b1, w2, b2, g1, be1, g2, be2,
                  yin1, yin2, bt, wp1, bp1, wp2, bp2,
                  y1, s1, q1, y2, s2, q2, out, accsc):
    i = pl.program_id(0)
    _c3_phase12(i, h0, h1, a0, a1, wa, wb, b1, w2, b2, g1, be1,
                yin1, y1, s1, q1, y2, s2, q2)

    @pl.when(i >= 2 * NP5)
    def _():
        a, c = _bn_coeffs(s2, q2, g2, be2)
        yb = yin2[...]
        y2[...] = yb
        h = jnp.maximum(yb * a + c, 0.0)
        onehot = (bt[...] == lax.broadcasted_iota(jnp.int32, (1, G), 1)
                  ).astype(F32)
        part = lax.dot_general(onehot, h, (((0,), (0,)), ((), ())),
                               preferred_element_type=F32)

        @pl.when(i == 2 * NP5)
        def _():
            accsc[...] = part

        @pl.when(i > 2 * NP5)
        def _():
            accsc[...] = accsc[...] + part

        @pl.when(i == 3 * NP5 - 1)
        def _():
            r = jnp.maximum(
                jnp.dot(accsc[...], wp1[...], preferred_element_type=F32)
                + bp1[...], 0.0)
            out[...] = (jnp.dot(r, wp2[...], preferred_element_type=F32)
                        + bp2[...])


def _make_c3(last):
    full = lambda r, c: pl.BlockSpec((r, c), lambda i: (0, 0))
    blk = lambda c, im: pl.BlockSpec((RB, c), lambda i: (im(i), 0))
    m_h = lambda i: jnp.minimum(i, NP5 - 1)
    m_yin1 = lambda i: jnp.where(i < NP5, NP5 - 1, jnp.minimum(i - NP5,
                                                              NP5 - 1))
    m_yin2 = lambda i: jnp.where(i < 2 * NP5, NP5 - 1, i - 2 * NP5)
    m_y1 = lambda i: jnp.where(i < NP5, i, jnp.minimum(i - NP5, NP5 - 1))
    m_y2 = lambda i: jnp.where(i < NP5, 0,
                               jnp.where(i < 2 * NP5, i - NP5, i - 2 * NP5))
    m_p3 = lambda i: jnp.maximum(i - 2 * NP5, 0)

    in_specs = [blk(128, m_h), blk(128, m_h), blk(128, m_h), blk(128, m_h),
                full(128, 256), full(128, 256), full(1, 256),
                full(256, 256), full(1, 256),
                full(1, 256), full(1, 256), full(1, 256), full(1, 256),
                blk(256, m_yin1), blk(256, m_yin2)]
    out_specs = [blk(256, m_y1), full(1, 256), full(1, 256),
                 blk(256, m_y2), full(1, 256), full(1, 256)]
    out_shape = [jax.ShapeDtypeStruct((NN, 256), F32),
                 jax.ShapeDtypeStruct((1, 256), F32),
                 jax.ShapeDtypeStruct((1, 256), F32),
                 jax.ShapeDtypeStruct((NN, 256), F32),
                 jax.ShapeDtypeStruct((1, 256), F32),
                 jax.ShapeDtypeStruct((1, 256), F32)]
    if last:
        body = _c3_last_body
        in_specs += [blk(1, m_p3), full(256, 256), full(1, 256),
                     full(256, NC), full(1, NC)]
        out_specs += [full(G, NC)]
        out_shape += [jax.ShapeDtypeStruct((G, NC), F32)]
        scratch = [pltpu.VMEM((G, 256), F32)]
    else:
        body = _c3_mid_body
        out_specs += [blk(128, m_p3), blk(128, m_p3)]
        out_shape += [jax.ShapeDtypeStruct((NN, 128), F32),
                      jax.ShapeDtypeStruct((NN, 128), F32)]
        scratch = []
    return pl.pallas_call(
        body,
        grid=(3 * NP5,),
        in_specs=in_specs,
        out_specs=out_specs,
        out_shape=out_shape,
        scratch_shapes=scratch,
        input_output_aliases={13: 0, 14: 3},
    )


def kernel(x, edge_index, batch, params):
    src = edge_index[0]
    dst = edge_index[1]
    e = src.shape[0]
    pad = EP - e
    pidx = jnp.arange(pad, dtype=jnp.int32) % 16
    src2 = jnp.concatenate([src, pidx]).reshape(-1, 128)
    dst2 = jnp.concatenate([dst, NN + pidx]).reshape(-1, 128)
    eb = jnp.stack([src2, dst2], axis=1)
    zeros_acc = jnp.zeros((ACC_ROWS, 128), F32)
    zplane = jnp.zeros((NN, 128), F32)

    sc_l1 = _make_sc_l1()
    sc_l23 = _make_sc_l23()
    c3_mid = _make_c3(False)
    c3_last = _make_c3(True)

    r1 = lambda t: t.reshape(1, -1)

    h0, h1 = x, zplane
    yb1 = jnp.zeros((NN, 256), F32)
    yb2 = yb1 + 1.0
    for li, p in enumerate(params['layers']):
        if li == 0:
            agg0, agg1 = sc_l1(h0, eb, zeros_acc)
            wa = wb = p['W1']
        else:
            agg0, agg1 = sc_l23(h0, h1, eb, zeros_acc)
            wa, wb = p['W1'][:128], p['W1'][128:]
        args = (h0, h1, agg0, agg1, wa, wb, r1(p['b1']), p['W2'], r1(p['b2']),
                r1(p['g1']), r1(p['be1']), r1(p['g2']), r1(p['be2']),
                yb1, yb2)
        if li < 2:
            y1o, _, _, y2o, _, _, h0, h1 = c3_mid(*args)
            yb1, yb2 = y1o, y2o
        else:
            outs = c3_last(*args,
                           batch.reshape(-1, 1).astype(jnp.int32),
                           params['Wp1'], r1(params['bp1']),
                           params['Wp2'], r1(params['bp2']))
    return outs[6]


# speedup vs baseline: 1.0465x; 1.0465x over previous
import functools

import jax
import jax.numpy as jnp
from jax import lax
from jax.experimental import pallas as pl
from jax.experimental.pallas import tpu as pltpu
from jax.experimental.pallas import tpu_sc as plsc

NN = 10000
ACC_ROWS = 10112
EP = 327680
LANES = 128
NTILES = 16
G = 64
NC = 10
RB = 2000
EPS_BN = 1e-5
F32 = jnp.float32


IDXC = 4
BSTEPS = 2 * IDXC


def _sc_edge_loop(h_ref, eb, row0, idx_a, idx_b, rows, acc, gsems, ssems,
                  isems, nrows):
    nbody = nrows // BSTEPS

    def drain_rows(sem, b):
        pltpu.make_async_copy(h_ref.at[pl.ds(0, LANES)], rows[b], sem).wait()

    def drain_idx(sem, ix):
        pltpu.make_async_copy(eb.at[pl.ds(0, IDXC)], ix, sem).wait()

    def body(base, first, last):
        if first:
            pltpu.sync_copy(eb.at[pl.ds(base, IDXC)], idx_a)
            pltpu.async_copy(h_ref.at[idx_a.at[0, 0]], rows[0], gsems[0])
        for j in range(BSTEPS):
            b = j % 2
            nb = 1 - b
            if not (first and j == 0):
                drain_rows(ssems[nb], nb)
            if j == 0:
                pltpu.async_copy(eb.at[pl.ds(base + IDXC, IDXC)], idx_b,
                                 isems[1])
            if j == IDXC and not last:
                pltpu.async_copy(eb.at[pl.ds(base + BSTEPS, IDXC)], idx_a,
                                 isems[0])
            if j + 1 < BSTEPS:
                nxt = j + 1
                if nxt == IDXC:
                    drain_idx(isems[1], idx_b)
                ix = idx_a if nxt < IDXC else idx_b
                pltpu.async_copy(h_ref.at[ix.at[nxt % IDXC, 0]], rows[nb],
                                 gsems[nb])
            elif not last:
                drain_idx(isems[0], idx_a)
                pltpu.async_copy(h_ref.at[idx_a.at[0, 0]], rows[nb],
                                 gsems[nb])
            drain_rows(gsems[b], b)
            ix = idx_a if j < IDXC else idx_b
            pltpu.async_copy(rows[b], acc.at[ix.at[j % IDXC, 1]], ssems[b],
                             add=True)
        if last:
            drain_rows(ssems[1], 1)

    body(row0, True, False)

    def mid(k2, carry):
        body(row0 + k2 * BSTEPS, False, False)
        return carry

    lax.fori_loop(1, nbody - 1, mid, 0)
    body(row0 + (nbody - 1) * BSTEPS, False, True)


def _sc_zero_acc(zeros_hbm, acc, s):
    zc = ACC_ROWS // NTILES
    pltpu.sync_copy(zeros_hbm.at[pl.ds(s * zc, zc)], acc.at[pl.ds(s * zc, zc)])
    plsc.subcore_barrier()


def _sc_copy_out(acc, agg0, agg1, c, s):
    plsc.subcore_barrier()
    oc = ACC_ROWS // NTILES

    @pl.when(c == 0)
    def _():
        pltpu.sync_copy(acc.at[pl.ds(s * oc, oc)], agg0.at[pl.ds(s * oc, oc)])

    @pl.when(c == 1)
    def _():
        pltpu.sync_copy(acc.at[pl.ds(s * oc, oc)], agg1.at[pl.ds(s * oc, oc)])


def _make_sc_l1():
    nrows = EP // 128 // (2 * NTILES)
    mesh = plsc.VectorSubcoreMesh(core_axis_name="c", subcore_axis_name="s")

    @functools.partial(
        pl.kernel,
        out_type=[jax.ShapeDtypeStruct((ACC_ROWS, 128), F32),
                  jax.ShapeDtypeStruct((ACC_ROWS, 128), F32)],
        mesh=mesh,
        scratch_types=[
            pltpu.VMEM((IDXC, 2, LANES), jnp.int32),
            pltpu.VMEM((IDXC, 2, LANES), jnp.int32),
            pltpu.VMEM((LANES, 128), F32),
            pltpu.VMEM((LANES, 128), F32),
            pltpu.VMEM_SHARED((ACC_ROWS, 128), F32),
            pltpu.SemaphoreType.DMA,
            pltpu.SemaphoreType.DMA,
            pltpu.SemaphoreType.DMA,
            pltpu.SemaphoreType.DMA,
            pltpu.SemaphoreType.DMA,
            pltpu.SemaphoreType.DMA,
        ],
    )
    def sc_l1(h_hbm, eb, zeros_hbm, agg0, agg1,
              idx_a, idx_b, rows_a, rows_b, acc,
              gs0, gs1, ss0, ss1, is0, is1):
        c = lax.axis_index("c")
        s = lax.axis_index("s")
        _sc_zero_acc(zeros_hbm, acc, s)
        row0 = (c * NTILES + s) * nrows
        _sc_edge_loop(h_hbm, eb, row0, idx_a, idx_b, (rows_a, rows_b), acc,
                      (gs0, gs1), (ss0, ss1), (is0, is1), nrows)
        _sc_copy_out(acc, agg0, agg1, c, s)

    return sc_l1


def _make_sc_l23():
    nrows = EP // 128 // NTILES
    mesh = plsc.VectorSubcoreMesh(core_axis_name="c", subcore_axis_name="s")

    @functools.partial(
        pl.kernel,
        out_type=[jax.ShapeDtypeStruct((ACC_ROWS, 128), F32),
                  jax.ShapeDtypeStruct((ACC_ROWS, 128), F32)],
        mesh=mesh,
        scratch_types=[
            pltpu.VMEM((IDXC, 2, LANES), jnp.int32),
            pltpu.VMEM((IDXC, 2, LANES), jnp.int32),
            pltpu.VMEM((LANES, 128), F32),
            pltpu.VMEM((LANES, 128), F32),
            pltpu.VMEM_SHARED((ACC_ROWS, 128), F32),
            pltpu.SemaphoreType.DMA,
            pltpu.SemaphoreType.DMA,
            pltpu.SemaphoreType.DMA,
            pltpu.SemaphoreType.DMA,
            pltpu.SemaphoreType.DMA,
            pltpu.SemaphoreType.DMA,
        ],
    )
    def sc_l23(h0_hbm, h1_hbm, eb, zeros_hbm, agg0, agg1,
               idx_a, idx_b, rows_a, rows_b, acc,
               gs0, gs1, ss0, ss1, is0, is1):
        c = lax.axis_index("c")
        s = lax.axis_index("s")
        _sc_zero_acc(zeros_hbm, acc, s)
        row0 = s * nrows

        @pl.when(c == 0)
        def _():
            _sc_edge_loop(h0_hbm, eb, row0, idx_a, idx_b, (rows_a, rows_b),
                          acc, (gs0, gs1), (ss0, ss1), (is0, is1), nrows)

        @pl.when(c == 1)
        def _():
            _sc_edge_loop(h1_hbm, eb, row0, idx_a, idx_b, (rows_a, rows_b),
                          acc, (gs0, gs1), (ss0, ss1), (is0, is1), nrows)

        _sc_copy_out(acc, agg0, agg1, c, s)

    return sc_l23


def _accum_stats(i, y_blk, s_ref, q_ref):
    s_p = jnp.sum(y_blk, axis=0, keepdims=True)
    q_p = jnp.sum(y_blk * y_blk, axis=0, keepdims=True)

    @pl.when(i == 0)
    def _():
        s_ref[...] = s_p
        q_ref[...] = q_p

    @pl.when(i > 0)
    def _():
        s_ref[...] = s_ref[...] + s_p
        q_ref[...] = q_ref[...] + q_p


def _bn_coeffs(s_ref, q_ref, g_ref, be_ref):
    m = s_ref[...] / NN
    v = q_ref[...] / NN - m * m
    a = g_ref[...] * lax.rsqrt(v + EPS_BN)
    c = be_ref[...] - m * a
    return a, c


NP5 = NN // RB


def _c3_phase12(i, h0, h1, a0, a1, wa, wb, b1, w2, b2, g1, be1,
                yin1, y1, s1, q1, y2, s2, q2):
    @pl.when(i < NP5)
    def _():
        xa = h0[...] + a0[...]
        xb = h1[...] + a1[...]
        blk = (jnp.dot(xa, wa[...], preferred_element_type=F32)
               + jnp.dot(xb, wb[...], preferred_element_type=F32)
               + b1[...])
        y1[...] = blk
        _accum_stats(i, blk, s1, q1)

    @pl.when(jnp.logical_and(i >= NP5, i < 2 * NP5))
    def _():
        a, c = _bn_coeffs(s1, q1, g1, be1)
        yb = yin1[...]
        x = jnp.maximum(yb * a + c, 0.0)
        blk = jnp.dot(x, w2[...], preferred_element_type=F32) + b2[...]
        y2[...] = blk
        _accum_stats(i - NP5, blk, s2, q2)


def _c3_mid_body(h0, h1, a0, a1, wa, wb, b1, w2, b2, g1, be1, g2, be2,
                 yin1, yin2, y1, s1, q1, y2, s2, q2, hp0, hp1):
    i = pl.program_id(0)
    _c3_phase12(i, h0, h1, a0, a1, wa, wb, b1, w2, b2, g1, be1,
                yin1, y1, s1, q1, y2, s2, q2)

    @pl.when(i >= 2 * NP5)
    def _():
        a, c = _bn_coeffs(s2, q2, g2, be2)
        yb = yin2[...]
        h = jnp.maximum(yb * a + c, 0.0)
        hp0[...] = h[:, :128]
        hp1[...] = h[:, 128:]


def _c3_last_body(h0, h1, a0, a1, wa, wb, b1, w2, b2, g1, be1, g2, be2,
                  yin1, yin2, bt, wp1, bp1, wp2, bp2,
                  y1, s1, q1, y2, s2, q2, out, accsc):
    i = pl.program_id(0)
    _c3_phase12(i, h0, h1, a0, a1, wa, wb, b1, w2, b2, g1, be1,
                yin1, y1, s1, q1, y2, s2, q2)

    @pl.when(i >= 2 * NP5)
    def _():
        a, c = _bn_coeffs(s2, q2, g2, be2)
        yb = yin2[...]
        h = jnp.maximum(yb * a + c, 0.0)
        onehot = (bt[...] == lax.broadcasted_iota(jnp.int32, (1, G), 1)
                  ).astype(F32)
        part = lax.dot_general(onehot, h, (((0,), (0,)), ((), ())),
                               preferred_element_type=F32)

        @pl.when(i == 2 * NP5)
        def _():
            accsc[...] = part

        @pl.when(i > 2 * NP5)
        def _():
            accsc[...] = accsc[...] + part

        @pl.when(i == 3 * NP5 - 1)
        def _():
            r = jnp.maximum(
                jnp.dot(accsc[...], wp1[...], preferred_element_type=F32)
                + bp1[...], 0.0)
            out[...] = (jnp.dot(r, wp2[...], preferred_element_type=F32)
                        + bp2[...])


def _make_c3(last):
    full = lambda r, c: pl.BlockSpec((r, c), lambda i: (0, 0))
    blk = lambda c, im: pl.BlockSpec((RB, c), lambda i: (im(i), 0))
    m_h = lambda i: jnp.minimum(i, NP5 - 1)
    m_yin1 = lambda i: jnp.where(i < NP5, NP5 - 1, jnp.minimum(i - NP5,
                                                              NP5 - 1))
    m_yin2 = lambda i: jnp.where(i < 2 * NP5, NP5 - 1, i - 2 * NP5)
    m_y1 = lambda i: jnp.minimum(i, NP5)
    m_y2 = lambda i: jnp.where(i < NP5, NP5,
                               jnp.where(i < 2 * NP5, i - NP5, NP5))
    m_p3 = lambda i: jnp.maximum(i - 2 * NP5, 0)

    in_specs = [blk(128, m_h), blk(128, m_h), blk(128, m_h), blk(128, m_h),
                full(128, 256), full(128, 256), full(1, 256),
                full(256, 256), full(1, 256),
                full(1, 256), full(1, 256), full(1, 256), full(1, 256),
                blk(256, m_yin1), blk(256, m_yin2)]
    out_specs = [blk(256, m_y1), full(1, 256), full(1, 256),
                 blk(256, m_y2), full(1, 256), full(1, 256)]
    out_shape = [jax.ShapeDtypeStruct((NN + RB, 256), F32),
                 jax.ShapeDtypeStruct((1, 256), F32),
                 jax.ShapeDtypeStruct((1, 256), F32),
                 jax.ShapeDtypeStruct((NN + RB, 256), F32),
                 jax.ShapeDtypeStruct((1, 256), F32),
                 jax.ShapeDtypeStruct((1, 256), F32)]
    if last:
        body = _c3_last_body
        in_specs += [blk(1, m_p3), full(256, 256), full(1, 256),
                     full(256, NC), full(1, NC)]
        out_specs += [full(G, NC)]
        out_shape += [jax.ShapeDtypeStruct((G, NC), F32)]
        scratch = [pltpu.VMEM((G, 256), F32)]
    else:
        body = _c3_mid_body
        out_specs += [blk(128, m_p3), blk(128, m_p3)]
        out_shape += [jax.ShapeDtypeStruct((NN, 128), F32),
                      jax.ShapeDtypeStruct((NN, 128), F32)]
        scratch = []
    return pl.pallas_call(
        body,
        grid=(3 * NP5,),
        in_specs=in_specs,
        out_specs=out_specs,
        out_shape=out_shape,
        scratch_shapes=scratch,
        input_output_aliases={13: 0, 14: 3},
    )


def kernel(x, edge_index, batch, params):
    src = edge_index[0]
    dst = edge_index[1]
    e = src.shape[0]
    pad = EP - e
    pidx = jnp.arange(pad, dtype=jnp.int32) % 16
    src2 = jnp.concatenate([src, pidx]).reshape(-1, 128)
    dst2 = jnp.concatenate([dst, NN + pidx]).reshape(-1, 128)
    eb = jnp.stack([src2, dst2], axis=1)
    zeros_acc = jnp.zeros((ACC_ROWS, 128), F32)
    zplane = jnp.zeros((NN, 128), F32)

    sc_l1 = _make_sc_l1()
    sc_l23 = _make_sc_l23()
    c3_mid = _make_c3(False)
    c3_last = _make_c3(True)

    r1 = lambda t: t.reshape(1, -1)

    h0, h1 = x, zplane
    yb1 = jnp.zeros((NN + RB, 256), F32)
    yb2 = yb1 + 1.0
    for li, p in enumerate(params['layers']):
        if li == 0:
            agg0, agg1 = sc_l1(h0, eb, zeros_acc)
            wa = wb = p['W1']
        else:
            agg0, agg1 = sc_l23(h0, h1, eb, zeros_acc)
            wa, wb = p['W1'][:128], p['W1'][128:]
        args = (h0, h1, agg0, agg1, wa, wb, r1(p['b1']), p['W2'], r1(p['b2']),
                r1(p['g1']), r1(p['be1']), r1(p['g2']), r1(p['be2']),
                yb1, yb2)
        if li < 2:
            y1o, _, _, y2o, _, _, h0, h1 = c3_mid(*args)
            yb1, yb2 = y1o, y2o
        else:
            outs = c3_last(*args,
                           batch.reshape(-1, 1).astype(jnp.int32),
                           params['Wp1'], r1(params['bp1']),
                           params['Wp2'], r1(params['bp2']))
    return outs[6]
